# baseline (speedup 1.0000x reference)
import functools

import jax
import jax.numpy as jnp
from jax.experimental import pallas as pl

_IN = 4
_NC = 16
_HID = 100
_HP = 128
_PAD = -100.0


def _fwd_kernel(xc_ref, w1_ref, w2_ref, out_ref, *, T):
    xc = xc_ref[0]
    pad = jnp.full((1, 3), _PAD, jnp.bfloat16)
    xe = jnp.concatenate([pad, xc], axis=1)

    xt = jnp.concatenate(
        [xe[:, 0:T], xe[:, 1:T + 1], xe[:, 2:T + 2], xe[:, 3:T + 3],
         jnp.ones((1, T), jnp.bfloat16)],
        axis=0,
    )

    ht = jnp.dot(w1_ref[...], xt, preferred_element_type=jnp.float32)
    ht = jnp.maximum(ht, 0.0)

    lt = jnp.dot(w2_ref[...], ht.astype(jnp.bfloat16),
                 preferred_element_type=jnp.float32)

    m = jnp.max(lt, axis=0, keepdims=True)
    s = jnp.sum(jnp.exp(lt - m), axis=0, keepdims=True)
    out_ref[0] = lt - m - jnp.log(s)


@functools.partial(jax.jit, static_argnames=("interpret",))
def kernel(input_, W1, b1, W2, b2, interpret=False):
    B, T = input_.shape

    w1a = jnp.zeros((_HP, _IN + 1), jnp.float32)
    w1a = w1a.at[:_HID, :_IN].set(W1.T).at[:_HID, _IN].set(b1)
    w1a = w1a.at[_HP - 1, _IN].set(1.0)
    w1a = w1a.astype(jnp.bfloat16)
    w2a = jnp.zeros((_NC, _HP), jnp.float32)
    w2a = w2a.at[:, :_HID].set(W2.T).at[:, _HP - 1].set(b2)
    w2a = w2a.astype(jnp.bfloat16)

    x3 = input_.astype(jnp.bfloat16).reshape(B, 1, T)

    out = pl.pallas_call(
        functools.partial(_fwd_kernel, T=T),
        grid=(B,),
        in_specs=[
            pl.BlockSpec((1, 1, T), lambda b: (b, 0, 0)),
            pl.BlockSpec((_HP, _IN + 1), lambda b: (0, 0)),
            pl.BlockSpec((_NC, _HP), lambda b: (0, 0)),
        ],
        out_specs=pl.BlockSpec((1, _NC, T), lambda b: (b, 0, 0)),
        out_shape=jax.ShapeDtypeStruct((B, _NC, T), jnp.float32),
        interpret=interpret,
    )(x3, w1a, w2a)
    return out.transpose(0, 2, 1)


# speedup vs baseline: 1.1133x; 1.1133x over previous
import functools

import jax
import jax.numpy as jnp
from jax.experimental import pallas as pl

_IN = 4
_NC = 16
_HID = 100
_HP = 128
_PAD = -100.0


def _fwd_kernel(xc_ref, w1_ref, w2_ref, out_ref, *, T, C):
    xc = xc_ref[0]
    pad = jnp.full((1, 3), _PAD, jnp.float32)
    xe = jnp.concatenate([pad, xc], axis=1)
    w1 = w1_ref[...]
    w2 = w2_ref[...]
    ones = jnp.ones((1, C), jnp.float32)

    for c0 in range(0, T, C):
        xt = jnp.concatenate(
            [xe[:, c0:c0 + C], xe[:, c0 + 1:c0 + C + 1],
             xe[:, c0 + 2:c0 + C + 2], xe[:, c0 + 3:c0 + C + 3], ones],
            axis=0,
        )

        ht = jnp.dot(w1, xt, preferred_element_type=jnp.float32)
        ht = jnp.maximum(ht, 0.0)

        lt = jnp.dot(w2, ht, preferred_element_type=jnp.float32)

        m = jnp.max(lt, axis=0, keepdims=True)
        s = jnp.sum(jnp.exp(lt - m), axis=0, keepdims=True)
        out_ref[0, :, c0:c0 + C] = lt - m - jnp.log(s)


@functools.partial(jax.jit, static_argnames=("interpret",))
def kernel(input_, W1, b1, W2, b2, interpret=False):
    B, T = input_.shape

    w1a = jnp.zeros((_HP, _IN + 1), jnp.float32)
    w1a = w1a.at[:_HID, :_IN].set(W1.T).at[:_HID, _IN].set(b1)
    w1a = w1a.at[_HP - 1, _IN].set(1.0)
    w2a = jnp.zeros((_NC, _HP), jnp.float32)
    w2a = w2a.at[:, :_HID].set(W2.T).at[:, _HP - 1].set(b2)

    x3 = input_.reshape(B, 1, T)

    out = pl.pallas_call(
        functools.partial(_fwd_kernel, T=T, C=4096),
        grid=(B,),
        in_specs=[
            pl.BlockSpec((1, 1, T), lambda b: (b, 0, 0)),
            pl.BlockSpec((_HP, _IN + 1), lambda b: (0, 0)),
            pl.BlockSpec((_NC, _HP), lambda b: (0, 0)),
        ],
        out_specs=pl.BlockSpec((1, _NC, T), lambda b: (b, 0, 0)),
        out_shape=jax.ShapeDtypeStruct((B, _NC, T), jnp.float32),
        interpret=interpret,
    )(x3, w1a, w2a)
    return out.transpose(0, 2, 1)


# speedup vs baseline: 1.3099x; 1.1765x over previous
import functools

import jax
import jax.numpy as jnp
from jax.experimental import pallas as pl

_IN = 4
_NC = 16
_HID = 100
_HP = 128
_PAD = -100.0


def _fwd_kernel(xc_ref, w1_ref, w2_ref, out_ref, *, T, C, R):
    w1 = w1_ref[...]
    w2 = w2_ref[...]
    pad = jnp.full((1, 3), _PAD, jnp.float32)
    ones = jnp.ones((1, C), jnp.float32)

    for r in range(R):
        xe = jnp.concatenate([pad, xc_ref[r]], axis=1)
        for c0 in range(0, T, C):
            xt = jnp.concatenate(
                [xe[:, c0:c0 + C], xe[:, c0 + 1:c0 + C + 1],
                 xe[:, c0 + 2:c0 + C + 2], xe[:, c0 + 3:c0 + C + 3], ones],
                axis=0,
            )

            ht = jnp.dot(w1, xt, preferred_element_type=jnp.float32)
            ht = jnp.maximum(ht, 0.0)

            lt = jnp.dot(w2, ht, preferred_element_type=jnp.float32)

            m = jnp.max(jnp.maximum(lt[:8, :], lt[8:, :]),
                        axis=0, keepdims=True)
            e = jnp.exp(lt - m)
            s = jnp.sum(e[:8, :] + e[8:, :], axis=0, keepdims=True)
            out_ref[r, :, c0:c0 + C] = lt - (m + jnp.log(s))


@functools.partial(jax.jit, static_argnames=("interpret",))
def kernel(input_, W1, b1, W2, b2, interpret=False):
    B, T = input_.shape

    w1a = jnp.zeros((_HP, _IN + 1), jnp.float32)
    w1a = w1a.at[:_HID, :_IN].set(W1.T).at[:_HID, _IN].set(b1)
    w1a = w1a.at[_HP - 1, _IN].set(1.0)
    w2a = jnp.zeros((_NC, _HP), jnp.float32)
    w2a = w2a.at[:, :_HID].set(W2.T).at[:, _HP - 1].set(b2)

    x3 = input_.reshape(B, 1, T)

    R = 2
    out = pl.pallas_call(
        functools.partial(_fwd_kernel, T=T, C=4096, R=R),
        grid=(B // R,),
        in_specs=[
            pl.BlockSpec((R, 1, T), lambda b: (b, 0, 0)),
            pl.BlockSpec((_HP, _IN + 1), lambda b: (0, 0)),
            pl.BlockSpec((_NC, _HP), lambda b: (0, 0)),
        ],
        out_specs=pl.BlockSpec((R, _NC, T), lambda b: (b, 0, 0)),
        out_shape=jax.ShapeDtypeStruct((B, _NC, T), jnp.float32),
        interpret=interpret,
    )(x3, w1a, w2a)
    return out.transpose(0, 2, 1)


# speedup vs baseline: 1.4366x; 1.0968x over previous
import functools

import jax
import jax.numpy as jnp
from jax.experimental import pallas as pl

_IN = 4
_NC = 16
_HID = 100
_HP = 128
_PAD = -100.0


def _fwd_kernel(xc_ref, w1_ref, w2_ref, out_ref, *, T, C, R):
    w1 = w1_ref[...]
    w2 = w2_ref[...]
    pad = jnp.full((1, 3), _PAD, jnp.float32)
    ones = jnp.ones((1, C), jnp.float32)

    for r in range(R):
        xe = jnp.concatenate([pad, xc_ref[r]], axis=1)
        for c0 in range(0, T, C):
            xt = jnp.concatenate(
                [xe[:, c0:c0 + C], xe[:, c0 + 1:c0 + C + 1],
                 xe[:, c0 + 2:c0 + C + 2], xe[:, c0 + 3:c0 + C + 3], ones],
                axis=0,
            )

            ht = jnp.dot(w1, xt, preferred_element_type=jnp.float32)
            ht = jnp.maximum(ht, 0.0)

            lt = jnp.dot(w2, ht, preferred_element_type=jnp.float32)

            m = jnp.max(jnp.maximum(lt[:8, :], lt[8:, :]),
                        axis=0, keepdims=True)
            e = jnp.exp(lt - m)
            s = jnp.sum(e[:8, :] + e[8:, :], axis=0, keepdims=True)
            out_ref[r, :, c0:c0 + C] = lt - (m + jnp.log(s))


@functools.partial(jax.jit, static_argnames=("interpret",))
def kernel(input_, W1, b1, W2, b2, interpret=False):
    B, T = input_.shape

    w1a = jnp.zeros((_HP, _IN + 1), jnp.float32)
    w1a = w1a.at[:_HID, :_IN].set(W1.T).at[:_HID, _IN].set(b1)
    w1a = w1a.at[_HP - 1, _IN].set(1.0)
    w2a = jnp.zeros((_NC, _HP), jnp.float32)
    w2a = w2a.at[:, :_HID].set(W2.T).at[:, _HP - 1].set(b2)

    x3 = input_.reshape(B, 1, T)

    R = 4
    out = pl.pallas_call(
        functools.partial(_fwd_kernel, T=T, C=4096, R=R),
        grid=(B // R,),
        in_specs=[
            pl.BlockSpec((R, 1, T), lambda b: (b, 0, 0)),
            pl.BlockSpec((_HP, _IN + 1), lambda b: (0, 0)),
            pl.BlockSpec((_NC, _HP), lambda b: (0, 0)),
        ],
        out_specs=pl.BlockSpec((R, _NC, T), lambda b: (b, 0, 0)),
        out_shape=jax.ShapeDtypeStruct((B, _NC, T), jnp.float32),
        interpret=interpret,
    )(x3, w1a, w2a)
    return out.transpose(0, 2, 1)


# speedup vs baseline: 1.5110x; 1.0518x over previous
import functools

import jax
import jax.numpy as jnp
from jax.experimental import pallas as pl

_IN = 4
_NC = 16
_HID = 100
_HP = 128
_PAD = -100.0


def _fwd_kernel(xc_ref, w1_ref, w2_ref, out_ref, *, T, C, R):
    w1 = w1_ref[...]
    w2 = w2_ref[...]
    pad = jnp.full((1, 3), _PAD, jnp.float32)
    ones = jnp.ones((1, C), jnp.float32)

    for r in range(R):
        xe = jnp.concatenate([pad, xc_ref[r]], axis=1)
        for c0 in range(0, T, C):
            xt = jnp.concatenate(
                [xe[:, c0:c0 + C], xe[:, c0 + 1:c0 + C + 1],
                 xe[:, c0 + 2:c0 + C + 2], xe[:, c0 + 3:c0 + C + 3], ones],
                axis=0,
            )

            ht = jnp.dot(w1, xt, preferred_element_type=jnp.float32)
            ht = jnp.maximum(ht, 0.0)

            lt = jnp.dot(w2, ht, preferred_element_type=jnp.float32)

            m = jnp.max(jnp.maximum(lt[:8, :], lt[8:, :]),
                        axis=0, keepdims=True)
            e = jnp.exp(lt - m)
            s = jnp.sum(e[:8, :] + e[8:, :], axis=0, keepdims=True)
            out_ref[r, :, c0:c0 + C] = lt - (m + jnp.log(s))


@functools.partial(jax.jit, static_argnames=("interpret",))
def kernel(input_, W1, b1, W2, b2, interpret=False):
    B, T = input_.shape

    w1a = jnp.zeros((_HP, _IN + 1), jnp.float32)
    w1a = w1a.at[:_HID, :_IN].set(W1.T).at[:_HID, _IN].set(b1)
    w1a = w1a.at[_HP - 1, _IN].set(1.0)
    w2a = jnp.zeros((_NC, _HP), jnp.float32)
    w2a = w2a.at[:, :_HID].set(W2.T).at[:, _HP - 1].set(b2)

    x3 = input_.reshape(B, 1, T)

    R = 8
    out = pl.pallas_call(
        functools.partial(_fwd_kernel, T=T, C=4096, R=R),
        grid=(B // R,),
        in_specs=[
            pl.BlockSpec((R, 1, T), lambda b: (b, 0, 0)),
            pl.BlockSpec((_HP, _IN + 1), lambda b: (0, 0)),
            pl.BlockSpec((_NC, _HP), lambda b: (0, 0)),
        ],
        out_specs=pl.BlockSpec((R, _NC, T), lambda b: (b, 0, 0)),
        out_shape=jax.ShapeDtypeStruct((B, _NC, T), jnp.float32),
        interpret=interpret,
    )(x3, w1a, w2a)
    return out.transpose(0, 2, 1)


# speedup vs baseline: 1.5356x; 1.0163x over previous
import functools

import jax
import jax.numpy as jnp
from jax.experimental import pallas as pl

_IN = 4
_NC = 16
_HID = 100
_HP = 128
_PAD = -100.0


def _fwd_kernel(xc_ref, w1_ref, w2_ref, out_ref, *, T, C, R):
    w1 = w1_ref[...]
    w2 = w2_ref[...]
    pad = jnp.full((1, 3), _PAD, jnp.float32)
    ones = jnp.ones((1, C), jnp.float32)

    for r in range(R):
        xe = jnp.concatenate([pad, xc_ref[r]], axis=1)
        for c0 in range(0, T, C):
            xt = jnp.concatenate(
                [xe[:, c0:c0 + C], xe[:, c0 + 1:c0 + C + 1],
                 xe[:, c0 + 2:c0 + C + 2], xe[:, c0 + 3:c0 + C + 3], ones],
                axis=0,
            )

            ht = jnp.dot(w1, xt, preferred_element_type=jnp.float32)
            ht = jnp.maximum(ht, 0.0)

            lt = jnp.dot(w2, ht, preferred_element_type=jnp.float32)

            m = jnp.max(jnp.maximum(lt[:8, :], lt[8:, :]),
                        axis=0, keepdims=True)
            e = jnp.exp(lt - m)
            s = jnp.sum(e[:8, :] + e[8:, :], axis=0, keepdims=True)
            out_ref[r, :, c0:c0 + C] = lt - (m + jnp.log(s))


@functools.partial(jax.jit, static_argnames=("interpret",))
def kernel(input_, W1, b1, W2, b2, interpret=False):
    B, T = input_.shape

    w1a = jnp.zeros((_HP, _IN + 1), jnp.float32)
    w1a = w1a.at[:_HID, :_IN].set(W1.T).at[:_HID, _IN].set(b1)
    w1a = w1a.at[_HP - 1, _IN].set(1.0)
    w2a = jnp.zeros((_NC, _HP), jnp.float32)
    w2a = w2a.at[:, :_HID].set(W2.T).at[:, _HP - 1].set(b2)

    x3 = input_.reshape(B, 1, T)

    R = 16
    out = pl.pallas_call(
        functools.partial(_fwd_kernel, T=T, C=4096, R=R),
        grid=(B // R,),
        in_specs=[
            pl.BlockSpec((R, 1, T), lambda b: (b, 0, 0)),
            pl.BlockSpec((_HP, _IN + 1), lambda b: (0, 0)),
            pl.BlockSpec((_NC, _HP), lambda b: (0, 0)),
        ],
        out_specs=pl.BlockSpec((R, _NC, T), lambda b: (b, 0, 0)),
        out_shape=jax.ShapeDtypeStruct((B, _NC, T), jnp.float32),
        interpret=interpret,
    )(x3, w1a, w2a)
    return out.transpose(0, 2, 1)
